# baseline (speedup 1.0000x reference)
import functools

import jax
import jax.numpy as jnp
from jax import lax
from jax.experimental import pallas as pl
from jax.experimental.pallas import tpu as pltpu
from jax.experimental.pallas import tpu_sc as plsc


def _make_gather_kernel(B, D, num_cores, num_subcores):
    NW = num_cores * num_subcores
    b_per_w = B // NW
    mesh = plsc.VectorSubcoreMesh(core_axis_name="c", subcore_axis_name="s")

    @functools.partial(
        pl.kernel,
        mesh=mesh,
        compiler_params=pltpu.CompilerParams(use_tc_tiling_on_sc=False),
        out_type=[
            jax.ShapeDtypeStruct((B, D), jnp.float32),
            jax.ShapeDtypeStruct((B, D), jnp.float32),
        ],
        scratch_types=[
            pltpu.VMEM((b_per_w,), jnp.int32),
            pltpu.VMEM((b_per_w,), jnp.int32),
            pltpu.VMEM((b_per_w, D), jnp.float32),
            pltpu.VMEM((b_per_w, D), jnp.float32),
            pltpu.SemaphoreType.DMA,
            pltpu.SemaphoreType.DMA,
        ],
    )
    def gather_kernel(uid_hbm, pid_hbm, ut_hbm, pt_hbm, uout_hbm, pout_hbm,
                      uidx_v, pidx_v, urows_v, prows_v, usem, psem):
        wid = lax.axis_index("s") * num_cores + lax.axis_index("c")
        base = wid * b_per_w
        pltpu.sync_copy(uid_hbm.at[pl.ds(base, b_per_w)], uidx_v)
        pltpu.sync_copy(pid_hbm.at[pl.ds(base, b_per_w)], pidx_v)
        ucopy = pltpu.async_copy(ut_hbm.at[uidx_v], urows_v, usem)
        pcopy = pltpu.async_copy(pt_hbm.at[pidx_v], prows_v, psem)
        ucopy.wait()
        pcopy.wait()
        pltpu.sync_copy(urows_v, uout_hbm.at[pl.ds(base, b_per_w)])
        pltpu.sync_copy(prows_v, pout_hbm.at[pl.ds(base, b_per_w)])

    return gather_kernel


def _mlp_body(ue_ref, pe_ref, w1u_ref, w1p_ref, b1_ref, w2_ref, b2_ref,
              w3_ref, b3_ref, out_ref):
    h = jnp.dot(ue_ref[...], w1u_ref[...], preferred_element_type=jnp.float32)
    h = h + jnp.dot(pe_ref[...], w1p_ref[...], preferred_element_type=jnp.float32)
    h = jnp.maximum(h + b1_ref[...], 0.0)
    h = jnp.dot(h, w2_ref[...], preferred_element_type=jnp.float32)
    h = jnp.maximum(h + b2_ref[...], 0.0)
    r = jnp.dot(h, w3_ref[...], preferred_element_type=jnp.float32)
    out_ref[...] = r + b3_ref[...]


def kernel(user_id, product_id, user_table, product_table, W1, b1, W2, b2, W3, b3):
    B = user_id.shape[0]
    D = user_table.shape[1]

    info = plsc.get_sparse_core_info()
    gather = _make_gather_kernel(B, D, info.num_cores, info.num_subcores)
    user_emb, product_emb = gather(user_id, product_id, user_table, product_table)

    BS = 1024
    grid = (B // BS,)
    w1u = W1[:D]
    w1p = W1[D:]
    b1r = b1.reshape(1, -1)
    b2r = b2.reshape(1, -1)
    b3r = b3.reshape(1, 1)

    rating = pl.pallas_call(
        _mlp_body,
        grid=grid,
        in_specs=[
            pl.BlockSpec((BS, D), lambda i: (i, 0)),
            pl.BlockSpec((BS, D), lambda i: (i, 0)),
            pl.BlockSpec(w1u.shape, lambda i: (0, 0)),
            pl.BlockSpec(w1p.shape, lambda i: (0, 0)),
            pl.BlockSpec(b1r.shape, lambda i: (0, 0)),
            pl.BlockSpec(W2.shape, lambda i: (0, 0)),
            pl.BlockSpec(b2r.shape, lambda i: (0, 0)),
            pl.BlockSpec(W3.shape, lambda i: (0, 0)),
            pl.BlockSpec(b3r.shape, lambda i: (0, 0)),
        ],
        out_specs=pl.BlockSpec((BS, 1), lambda i: (i, 0)),
        out_shape=jax.ShapeDtypeStruct((B, 1), jnp.float32),
    )(user_emb, product_emb, w1u, w1p, b1r, W2, b2r, W3, b3r)

    return (user_emb, product_emb, rating)


# speedup vs baseline: 1.4135x; 1.4135x over previous
import functools

import jax
import jax.numpy as jnp
from jax import lax
from jax.experimental import pallas as pl
from jax.experimental.pallas import tpu as pltpu
from jax.experimental.pallas import tpu_sc as plsc

B = 4096
V = 100001
D = 32
CW = 3200
TAIL0 = 99968
LASTLO = 96768
OUTROWS = B + 128
PACK = 4096


def _make_sc_gather(num_cores, num_subcores):
    NW = num_cores * num_subcores
    assert (NW - 1) * CW < TAIL0 <= LASTLO + CW
    mesh = plsc.VectorSubcoreMesh(core_axis_name="c", subcore_axis_name="s")

    @functools.partial(
        pl.kernel,
        mesh=mesh,
        compiler_params=pltpu.CompilerParams(
            use_tc_tiling_on_sc=True, needs_layout_passes=False),
        out_type=[
            jax.ShapeDtypeStruct((OUTROWS, 128), jnp.float32),
            jax.ShapeDtypeStruct((OUTROWS, 128), jnp.float32),
        ],
        scratch_types=[
            pltpu.VMEM((D, CW), jnp.float32),
            pltpu.VMEM((B,), jnp.int32),
            pltpu.VMEM((B,), jnp.int32),
            pltpu.VMEM((128,), jnp.int32),
            pltpu.VMEM((128,), jnp.int32),
            pltpu.VMEM((128, 128), jnp.float32),
            pltpu.SemaphoreType.DMA,
            pltpu.SemaphoreType.DMA,
            pltpu.SemaphoreType.DMA,
        ],
    )
    def sc_gather(uid_h, pid_h, ut_h, pt_h, tu_h, tp_h, ou_h, op_h,
                  chunk, uidx, pidx, cval, cpos, staging,
                  csem, isem, ssem):
        cix = lax.axis_index("c")
        six = lax.axis_index("s")
        wid = six * num_cores + cix
        lo = wid * CW
        col_lo = jnp.minimum(lo, LASTLO)
        dma_lo = pl.multiple_of(col_lo, 128)

        iota16 = lax.iota(jnp.int32, 16)
        trash16 = 4096 + ((iota16 * 8 + wid) & 127)
        trash_packed = trash16 * PACK

        icp_u = pltpu.async_copy(uid_h, uidx, isem)
        icp_p = pltpu.async_copy(pid_h, pidx, isem)
        icp_u.wait()
        icp_p.wait()

        def prefill():
            for g in range(8):
                cval[pl.ds(g * 16, 16)] = trash_packed

        def gather_scatter(out_hbm, src_ref, src_cols):
            for g in range(8):
                val16 = cval[pl.ds(g * 16, 16)]
                pos16 = lax.shift_right_logical(val16, 12)
                cpos[pl.ds(g * 16, 16)] = pos16
                q16 = val16 & (PACK - 1)
                for e in range(16):
                    qv = jnp.broadcast_to(q16[e], (16,))
                    r0 = plsc.load_gather(src_ref, [iota16, qv])
                    r1 = plsc.load_gather(src_ref, [iota16 + 16, qv])
                    staging[g * 16 + e, pl.ds(0, 16)] = r0
                    staging[g * 16 + e, pl.ds(16, 16)] = r1
            pltpu.async_copy(staging, out_hbm.at[cpos], ssem).wait()

        def do_table(idx_ref, t_hbm, out_hbm):
            ccp = pltpu.async_copy(
                t_hbm.at[:, pl.ds(dma_lo, CW)], chunk, csem)
            ccp.wait()
            prefill()

            def scan_step(j, cnt):
                v16 = idx_ref[pl.ds(j * 16, 16)]
                m = (v16 >= lo) & (v16 < lo + CW) & (v16 < TAIL0)
                key = 1 - m.astype(jnp.int32)
                q16 = v16 - col_lo
                pos16 = iota16 + j * 16
                val = jnp.where(m, pos16 * PACK + q16, trash_packed)
                _, vs = plsc.sort_key_val(key, val)
                cval[pl.ds(cnt, 16)] = vs
                new_cnt = cnt + jnp.sum(m.astype(jnp.int32))

                @pl.when(new_cnt >= 112)
                def _():
                    gather_scatter(out_hbm, chunk, CW)

                return jnp.where(new_cnt >= 112, 0, new_cnt)

            lax.fori_loop(0, B // 16, scan_step, jnp.int32(0))
            gather_scatter(out_hbm, chunk, CW)

        do_table(uidx, ut_h, ou_h)
        do_table(pidx, pt_h, op_h)

        def do_tail(idx_ref, tail_hbm, out_hbm):
            tcp = pltpu.async_copy(tail_hbm, chunk.at[:, pl.ds(0, 128)], csem)
            tcp.wait()
            prefill()

            def scan_step(j, cnt):
                base = wid * 128 + j * 16
                v16 = idx_ref[pl.ds(base, 16)]
                m = v16 >= TAIL0
                key = 1 - m.astype(jnp.int32)
                q16 = v16 - TAIL0
                pos16 = iota16 + base
                val = jnp.where(m, pos16 * PACK + q16, trash_packed)
                _, vs = plsc.sort_key_val(key, val)
                cval[pl.ds(cnt, 16)] = vs
                return cnt + jnp.sum(m.astype(jnp.int32))

            cnt = lax.fori_loop(0, 8, scan_step, jnp.int32(0))

            @pl.when(cnt > 0)
            def _():
                gather_scatter(out_hbm, chunk, CW)

        do_tail(uidx, tu_h, ou_h)
        do_tail(pidx, tp_h, op_h)

    return sc_gather


def _mlp_body(ue_ref, pe_ref, w1u_ref, w1p_ref, b1_ref, w2_ref, b2_ref,
              w3_ref, b3_ref, out_ref):
    xu = ue_ref[...][:, :D]
    xp = pe_ref[...][:, :D]
    h = jnp.dot(xu, w1u_ref[...], preferred_element_type=jnp.float32)
    h = h + jnp.dot(xp, w1p_ref[...], preferred_element_type=jnp.float32)
    h = jnp.maximum(h + b1_ref[...], 0.0)
    h = jnp.dot(h, w2_ref[...], preferred_element_type=jnp.float32)
    h = jnp.maximum(h + b2_ref[...], 0.0)
    r = jnp.dot(h, w3_ref[...], preferred_element_type=jnp.float32)
    out_ref[...] = r + b3_ref[...]


def kernel(user_id, product_id, user_table, product_table, W1, b1, W2, b2, W3, b3):
    uT = user_table.T
    pT = product_table.T
    tail_u = jnp.pad(lax.slice(uT, (0, TAIL0), (D, V)), ((0, 0), (0, 128 - (V - TAIL0))))
    tail_p = jnp.pad(lax.slice(pT, (0, TAIL0), (D, V)), ((0, 0), (0, 128 - (V - TAIL0))))

    info = plsc.get_sparse_core_info()
    sc_gather = _make_sc_gather(info.num_cores, info.num_subcores)
    ou, op = sc_gather(user_id, product_id, uT, pT, tail_u, tail_p)

    user_emb = lax.slice(ou, (0, 0), (B, D))
    product_emb = lax.slice(op, (0, 0), (B, D))

    BS = 1024
    w1u = W1[:D]
    w1p = W1[D:]
    b1r = b1.reshape(1, -1)
    b2r = b2.reshape(1, -1)
    b3r = b3.reshape(1, 1)

    rating = pl.pallas_call(
        _mlp_body,
        grid=(B // BS,),
        in_specs=[
            pl.BlockSpec((BS, 128), lambda i: (i, 0)),
            pl.BlockSpec((BS, 128), lambda i: (i, 0)),
            pl.BlockSpec(w1u.shape, lambda i: (0, 0)),
            pl.BlockSpec(w1p.shape, lambda i: (0, 0)),
            pl.BlockSpec(b1r.shape, lambda i: (0, 0)),
            pl.BlockSpec(W2.shape, lambda i: (0, 0)),
            pl.BlockSpec(b2r.shape, lambda i: (0, 0)),
            pl.BlockSpec(W3.shape, lambda i: (0, 0)),
            pl.BlockSpec(b3r.shape, lambda i: (0, 0)),
        ],
        out_specs=pl.BlockSpec((BS, 1), lambda i: (i, 0)),
        out_shape=jax.ShapeDtypeStruct((B, 1), jnp.float32),
    )(ou, op, w1u, w1p, b1r, W2, b2r, W3, b3r)

    return (user_emb, product_emb, rating)


# speedup vs baseline: 1.7495x; 1.2377x over previous
import functools

import jax
import jax.numpy as jnp
from jax import lax
from jax.experimental import pallas as pl
from jax.experimental.pallas import tpu as pltpu
from jax.experimental.pallas import tpu_sc as plsc

B = 4096
V = 100001
D = 32
CW = 3200
TAIL0 = 99968
LASTLO = 96768
OUTROWS = B + 128
PACK = 4096


def _make_sc_gather(num_cores, num_subcores):
    NW = num_cores * num_subcores
    assert (NW - 1) * CW < TAIL0 <= LASTLO + CW
    mesh = plsc.VectorSubcoreMesh(core_axis_name="c", subcore_axis_name="s")

    @functools.partial(
        pl.kernel,
        mesh=mesh,
        compiler_params=pltpu.CompilerParams(
            use_tc_tiling_on_sc=True, needs_layout_passes=False),
        out_type=[
            jax.ShapeDtypeStruct((OUTROWS, 128), jnp.float32),
            jax.ShapeDtypeStruct((OUTROWS, 128), jnp.float32),
        ],
        scratch_types=[
            pltpu.VMEM((D, CW), jnp.float32),
            pltpu.VMEM((B,), jnp.int32),
            pltpu.VMEM((B,), jnp.int32),
            pltpu.VMEM((128,), jnp.int32),
            pltpu.VMEM((128,), jnp.int32),
            pltpu.VMEM((128, 128), jnp.float32),
            pltpu.SMEM((1,), jnp.int32),
            pltpu.SemaphoreType.DMA,
            pltpu.SemaphoreType.DMA,
            pltpu.SemaphoreType.DMA,
        ],
    )
    def sc_gather(uid_h, pid_h, ut_h, pt_h, tu_h, tp_h, ou_h, op_h,
                  chunk, uidx, pidx, cval, cpos, staging, waited,
                  csem, isem, ssem):
        cix = lax.axis_index("c")
        six = lax.axis_index("s")
        wid = six * num_cores + cix
        lo = wid * CW
        col_lo = jnp.minimum(lo, LASTLO)
        dma_lo = pl.multiple_of(col_lo, 128)

        iota16 = lax.iota(jnp.int32, 16)
        posP = iota16 * PACK
        trash16 = 4096 + ((iota16 * 8 + wid) & 127)
        trash_packed = trash16 * PACK

        icp_u = pltpu.async_copy(uid_h, uidx, isem)
        icp_p = pltpu.async_copy(pid_h, pidx, isem)
        icp_u.wait()
        icp_p.wait()

        def prefill():
            for g in range(8):
                cval[pl.ds(g * 16, 16)] = trash_packed

        def wait_chunk(t_hbm):
            @pl.when(waited[0] == 0)
            def _():
                pltpu.make_async_copy(
                    t_hbm.at[:, pl.ds(dma_lo, CW)], chunk, csem).wait()
                waited[0] = 1

        def gather_scatter(out_hbm, t_hbm):
            wait_chunk(t_hbm)

            def per_group(g, _):
                val16 = cval[pl.ds(g * 16, 16)]
                pos16 = lax.shift_right_logical(val16, 12)
                cpos[pl.ds(g * 16, 16)] = pos16
                q16 = val16 & (PACK - 1)
                for e in range(16):
                    qv = jnp.broadcast_to(q16[e], (16,))
                    r0 = plsc.load_gather(chunk, [iota16, qv])
                    r1 = plsc.load_gather(chunk, [iota16 + 16, qv])
                    staging[g * 16 + e, pl.ds(0, 16)] = r0
                    staging[g * 16 + e, pl.ds(16, 16)] = r1
                return 0

            lax.fori_loop(0, 8, per_group, 0)
            pltpu.async_copy(staging, out_hbm.at[cpos], ssem).wait()

        def do_table(idx_ref, t_hbm, out_hbm):
            waited[0] = 0
            ccp = pltpu.async_copy(
                t_hbm.at[:, pl.ds(dma_lo, CW)], chunk, csem)
            del ccp
            prefill()

            def scan_step(j, cnt):
                v16 = idx_ref[pl.ds(j * 16, 16)]
                m = (v16 >= lo) & (v16 < lo + CW) & (v16 < TAIL0)
                key = 1 - m.astype(jnp.int32)
                q16 = v16 - col_lo
                val = jnp.where(m, posP + j * (16 * PACK) + q16, trash_packed)
                _, vs = plsc.sort_key_val(key, val)
                cval[pl.ds(cnt, 16)] = vs
                new_cnt = cnt + plsc.all_reduce_population_count(m)[0]

                @pl.when(new_cnt >= 112)
                def _():
                    gather_scatter(out_hbm, t_hbm)
                    prefill()

                return jnp.where(new_cnt >= 112, 0, new_cnt)

            lax.fori_loop(0, B // 16, scan_step, jnp.int32(0), unroll=4)
            gather_scatter(out_hbm, t_hbm)

        do_table(uidx, ut_h, ou_h)
        do_table(pidx, pt_h, op_h)

        def do_tail(idx_ref, tail_hbm, out_hbm):
            pltpu.async_copy(tail_hbm, chunk.at[:, pl.ds(0, 128)], csem).wait()
            waited[0] = 1
            prefill()

            def scan_step(j, cnt):
                base = wid * 128 + j * 16
                v16 = idx_ref[pl.ds(base, 16)]
                m = v16 >= TAIL0
                key = 1 - m.astype(jnp.int32)
                q16 = v16 - TAIL0
                val = jnp.where(m, posP + base * PACK + q16, trash_packed)
                _, vs = plsc.sort_key_val(key, val)
                cval[pl.ds(cnt, 16)] = vs
                return cnt + plsc.all_reduce_population_count(m)[0]

            cnt = lax.fori_loop(0, 8, scan_step, jnp.int32(0))

            @pl.when(cnt > 0)
            def _():
                gather_scatter(out_hbm, tail_hbm)

        do_tail(uidx, tu_h, ou_h)
        do_tail(pidx, tp_h, op_h)

    return sc_gather


def _mlp_body(ue_ref, pe_ref, w1u_ref, w1p_ref, b1_ref, w2_ref, b2_ref,
              w3_ref, b3_ref, out_ref, etu_ref, etp_ref):
    xu = ue_ref[...][:, :D]
    xp = pe_ref[...][:, :D]
    etu_ref[...] = xu.T
    etp_ref[...] = xp.T
    h = jnp.dot(xu, w1u_ref[...], preferred_element_type=jnp.float32)
    h = h + jnp.dot(xp, w1p_ref[...], preferred_element_type=jnp.float32)
    h = jnp.maximum(h + b1_ref[...], 0.0)
    h = jnp.dot(h, w2_ref[...], preferred_element_type=jnp.float32)
    h = jnp.maximum(h + b2_ref[...], 0.0)
    r = jnp.dot(h, w3_ref[...], preferred_element_type=jnp.float32)
    out_ref[...] = r + b3_ref[...]


def kernel(user_id, product_id, user_table, product_table, W1, b1, W2, b2, W3, b3):
    uT = user_table.T
    pT = product_table.T
    tail_u = jnp.pad(lax.slice(uT, (0, TAIL0), (D, V)),
                     ((0, 0), (0, 128 - (V - TAIL0))))
    tail_p = jnp.pad(lax.slice(pT, (0, TAIL0), (D, V)),
                     ((0, 0), (0, 128 - (V - TAIL0))))

    info = plsc.get_sparse_core_info()
    sc_gather = _make_sc_gather(info.num_cores, info.num_subcores)
    ou, op = sc_gather(user_id, product_id, uT, pT, tail_u, tail_p)

    BS = 1024
    w1u = W1[:D]
    w1p = W1[D:]
    b1r = b1.reshape(1, -1)
    b2r = b2.reshape(1, -1)
    b3r = b3.reshape(1, 1)

    rating, embTu, embTp = pl.pallas_call(
        _mlp_body,
        grid=(B // BS,),
        in_specs=[
            pl.BlockSpec((BS, 128), lambda i: (i, 0)),
            pl.BlockSpec((BS, 128), lambda i: (i, 0)),
            pl.BlockSpec(w1u.shape, lambda i: (0, 0)),
            pl.BlockSpec(w1p.shape, lambda i: (0, 0)),
            pl.BlockSpec(b1r.shape, lambda i: (0, 0)),
            pl.BlockSpec(W2.shape, lambda i: (0, 0)),
            pl.BlockSpec(b2r.shape, lambda i: (0, 0)),
            pl.BlockSpec(W3.shape, lambda i: (0, 0)),
            pl.BlockSpec(b3r.shape, lambda i: (0, 0)),
        ],
        out_specs=[
            pl.BlockSpec((BS, 1), lambda i: (i, 0)),
            pl.BlockSpec((D, BS), lambda i: (0, i)),
            pl.BlockSpec((D, BS), lambda i: (0, i)),
        ],
        out_shape=[
            jax.ShapeDtypeStruct((B, 1), jnp.float32),
            jax.ShapeDtypeStruct((D, B), jnp.float32),
            jax.ShapeDtypeStruct((D, B), jnp.float32),
        ],
    )(ou, op, w1u, w1p, b1r, W2, b2r, W3, b3r)

    return (embTu.T, embTp.T, rating)
